# baseline (speedup 1.0000x reference)
import functools

import jax
import jax.numpy as jnp
from jax import lax
from jax.experimental import pallas as pl
from jax.experimental.pallas import tpu as pltpu
from jax.experimental.pallas import tpu_sc as plsc

POOL = 7
NSAMP = POOL * POOL
KPAD = 64
GROWS = 56
NC, NS, LANES = 2, 16, 16
NW = NC * NS


def _log2(x):
    return jnp.log(x) / jnp.log(2.0)


def _prep(boxes, image_shape, sizes):
    f32 = jnp.float32
    N = boxes.shape[0] * boxes.shape[1]
    fb = boxes.reshape(N, 4)
    y1 = fb[:, 0]
    x1 = fb[:, 1]
    y2 = fb[:, 2]
    x2 = fb[:, 3]
    h = y2 - y1
    w = x2 - x1
    image_area = (image_shape[0] * image_shape[1]).astype(f32)
    roi_level = _log2(jnp.sqrt(h * w) / (224.0 / jnp.sqrt(image_area)))
    roi_level = jnp.minimum(
        5, jnp.maximum(2, 4 + jnp.round(roi_level).astype(jnp.int32))
    )

    ar = jnp.arange(POOL, dtype=f32)[None, :]
    sel_ly = jnp.zeros((N, POOL), f32)
    sel_lx = jnp.zeros((N, POOL), f32)
    sel_ry0 = jnp.zeros((N, POOL), jnp.int32)
    sel_ry1 = jnp.zeros((N, POOL), jnp.int32)
    sel_cx0 = jnp.zeros((N, POOL), jnp.int32)
    sel_cx1 = jnp.zeros((N, POOL), jnp.int32)
    for li, H in enumerate(sizes):
        level = li + 2
        W = H
        ys = y1[:, None] * (H - 1) + ar * ((y2 - y1)[:, None] * (H - 1) / (POOL - 1))
        xs = x1[:, None] * (W - 1) + ar * ((x2 - x1)[:, None] * (W - 1) / (POOL - 1))
        y0f = jnp.floor(ys)
        x0f = jnp.floor(xs)
        y0 = jnp.clip(y0f.astype(jnp.int32), 0, H - 1)
        y1i = jnp.clip(y0 + 1, 0, H - 1)
        x0 = jnp.clip(x0f.astype(jnp.int32), 0, W - 1)
        x1c = jnp.clip(x0 + 1, 0, W - 1)
        ly = ys - y0f
        lx = xs - x0f
        m = (roi_level == level)[:, None]
        sel_ly = jnp.where(m, ly, sel_ly)
        sel_lx = jnp.where(m, lx, sel_lx)
        sel_ry0 = jnp.where(m, y0 * W, sel_ry0)
        sel_ry1 = jnp.where(m, y1i * W, sel_ry1)
        sel_cx0 = jnp.where(m, x0, sel_cx0)
        sel_cx1 = jnp.where(m, x1c, sel_cx1)

    def cross(ry, cx):
        return (ry[:, :, None] + cx[:, None, :]).reshape(N, NSAMP)

    idx4 = jnp.stack(
        [cross(sel_ry0, sel_cx0), cross(sel_ry0, sel_cx1),
         cross(sel_ry1, sel_cx0), cross(sel_ry1, sel_cx1)], axis=1
    )
    idx4 = jnp.pad(idx4, ((0, 0), (0, 0), (0, KPAD - NSAMP)))
    idx4 = idx4.reshape(N, 4 * KPAD)
    ly49 = jnp.broadcast_to(sel_ly[:, :, None], (N, POOL, POOL)).reshape(N, NSAMP)
    lx49 = jnp.broadcast_to(sel_lx[:, None, :], (N, POOL, POOL)).reshape(N, NSAMP)
    zpad = jnp.zeros((N, KPAD - NSAMP), f32)
    wrec = jnp.concatenate([ly49, zpad, lx49, zpad], axis=1)
    return idx4, wrec, roi_level


def _make_sc_call(N, C):
    CCH = C // LANES
    BPW = (N + NW - 1) // NW
    mesh = plsc.VectorSubcoreMesh(
        core_axis_name="c", subcore_axis_name="s", num_cores=NC, num_subcores=NS
    )

    @functools.partial(
        pl.kernel,
        out_type=jax.ShapeDtypeStruct((1, N, POOL, POOL, C), jnp.float32),
        mesh=mesh,
        compiler_params=pltpu.CompilerParams(use_tc_tiling_on_sc=True),
        scratch_types=[
            pltpu.VMEM((128,), jnp.int32),
            pltpu.VMEM((BPW, 4 * KPAD), jnp.int32),
            pltpu.VMEM((BPW, 2 * KPAD), jnp.float32),
            pltpu.VMEM((4, GROWS, C), jnp.float32),
            pltpu.VMEM((POOL, POOL, C), jnp.float32),
            pltpu.SemaphoreType.DMA,
        ],
    )
    def roialign_sc(t2, t3, t4, t5, idx_hbm, w_hbm, lvl_hbm, out_hbm,
                    lvl_v, idx_all, w_all, rows_v, out_v, sem):
        wid = lax.axis_index("s") * NC + lax.axis_index("c")
        start = wid * BPW
        nb = jnp.clip(N - start, 0, BPW)
        pltpu.sync_copy(lvl_hbm.at[wid], lvl_v)
        pltpu.sync_copy(idx_hbm.at[pl.ds(start, BPW)], idx_all)
        pltpu.sync_copy(w_hbm.at[pl.ds(start, BPW)], w_all)

        def box_body(b, carry):
            n = start + b
            lvl = lvl_v[pl.ds(b, LANES)][0]
            for level, tref in ((2, t2), (3, t3), (4, t4), (5, t5)):
                def gather(tref=tref):
                    cps = [
                        pltpu.async_copy(
                            tref.at[idx_all.at[b, pl.ds(k * KPAD, GROWS)]],
                            rows_v.at[k], sem)
                        for k in range(4)
                    ]
                    for cp in cps:
                        cp.wait()
                pl.when(lvl == level)(gather)

            def samp(j, c2):
                ly_s = w_all[b, pl.ds(j, LANES)][0]
                lx_s = w_all[b, pl.ds(KPAD + j, LANES)][0]
                iy = j // POOL
                ix = j % POOL
                for c in range(CCH):
                    s = pl.ds(c * LANES, LANES)
                    v00 = rows_v[0, j, s]
                    v01 = rows_v[1, j, s]
                    v10 = rows_v[2, j, s]
                    v11 = rows_v[3, j, s]
                    top = v00 + (v01 - v00) * lx_s
                    bot = v10 + (v11 - v10) * lx_s
                    out_v[iy, ix, s] = top + (bot - top) * ly_s
                return c2

            lax.fori_loop(0, NSAMP, samp, 0)
            pltpu.sync_copy(out_v, out_hbm.at[0, n])
            return carry

        lax.fori_loop(0, nb, box_body, 0)

    return roialign_sc


def kernel(boxes, image_shape, P2, P3, P4, P5):
    B, N = boxes.shape[0], boxes.shape[1]
    C = P2.shape[-1]
    sizes = (P2.shape[1], P3.shape[1], P4.shape[1], P5.shape[1])
    idx4, wrec, lvl = _prep(boxes, image_shape, sizes)
    BPW = (B * N + NW - 1) // NW
    nslots = NW * BPW
    idx_pad = jnp.pad(idx4, ((0, nslots - B * N), (0, 0)))
    w_pad = jnp.pad(wrec, ((0, nslots - B * N), (0, 0)))
    lvl_2d = jnp.pad(lvl, (0, nslots - B * N), constant_values=2)
    lvl_2d = jnp.pad(lvl_2d.reshape(NW, BPW), ((0, 0), (0, 128 - BPW)))
    tables = [p.reshape(-1, C) for p in (P2, P3, P4, P5)]
    out = _make_sc_call(B * N, C)(*tables, idx_pad, w_pad, lvl_2d)
    return out


# speedup vs baseline: 1.6423x; 1.6423x over previous
import functools

import jax
import jax.numpy as jnp
from jax import lax
from jax.experimental import pallas as pl
from jax.experimental.pallas import tpu as pltpu
from jax.experimental.pallas import tpu_sc as plsc

POOL = 7
NSAMP = POOL * POOL
KPAD = 64
NC, NS, LANES = 2, 16, 16
NW = NC * NS


def _log2(x):
    return jnp.log(x) / jnp.log(2.0)


def _prep(boxes, image_shape, sizes):
    f32 = jnp.float32
    N = boxes.shape[0] * boxes.shape[1]
    fb = boxes.reshape(N, 4)
    y1 = fb[:, 0]
    x1 = fb[:, 1]
    y2 = fb[:, 2]
    x2 = fb[:, 3]
    h = y2 - y1
    w = x2 - x1
    image_area = (image_shape[0] * image_shape[1]).astype(f32)
    roi_level = _log2(jnp.sqrt(h * w) / (224.0 / jnp.sqrt(image_area)))
    roi_level = jnp.minimum(
        5, jnp.maximum(2, 4 + jnp.round(roi_level).astype(jnp.int32))
    )

    ar = jnp.arange(POOL, dtype=f32)[None, :]
    sel_ly = jnp.zeros((N, POOL), f32)
    sel_lx = jnp.zeros((N, POOL), f32)
    sel_ry0 = jnp.zeros((N, POOL), jnp.int32)
    sel_ry1 = jnp.zeros((N, POOL), jnp.int32)
    sel_cx0 = jnp.zeros((N, POOL), jnp.int32)
    sel_cx1 = jnp.zeros((N, POOL), jnp.int32)
    for li, H in enumerate(sizes):
        level = li + 2
        W = H
        ys = y1[:, None] * (H - 1) + ar * ((y2 - y1)[:, None] * (H - 1) / (POOL - 1))
        xs = x1[:, None] * (W - 1) + ar * ((x2 - x1)[:, None] * (W - 1) / (POOL - 1))
        y0f = jnp.floor(ys)
        x0f = jnp.floor(xs)
        y0 = jnp.clip(y0f.astype(jnp.int32), 0, H - 1)
        y1i = jnp.clip(y0 + 1, 0, H - 1)
        x0 = jnp.clip(x0f.astype(jnp.int32), 0, W - 1)
        x1c = jnp.clip(x0 + 1, 0, W - 1)
        ly = ys - y0f
        lx = xs - x0f
        m = (roi_level == level)[:, None]
        sel_ly = jnp.where(m, ly, sel_ly)
        sel_lx = jnp.where(m, lx, sel_lx)
        sel_ry0 = jnp.where(m, y0 * W, sel_ry0)
        sel_ry1 = jnp.where(m, y1i * W, sel_ry1)
        sel_cx0 = jnp.where(m, x0, sel_cx0)
        sel_cx1 = jnp.where(m, x1c, sel_cx1)

    def cross(ry, cx):
        return (ry[:, :, None] + cx[:, None, :]).reshape(N, NSAMP)

    idx4 = jnp.stack(
        [cross(sel_ry0, sel_cx0), cross(sel_ry0, sel_cx1),
         cross(sel_ry1, sel_cx0), cross(sel_ry1, sel_cx1)], axis=1
    )
    ly49 = jnp.broadcast_to(sel_ly[:, :, None], (N, POOL, POOL)).reshape(N, NSAMP)
    lx49 = jnp.broadcast_to(sel_lx[:, None, :], (N, POOL, POOL)).reshape(N, NSAMP)
    zpad = jnp.zeros((N, KPAD - NSAMP), f32)
    wrec = jnp.concatenate([ly49, zpad, lx49, zpad], axis=1)
    return idx4, wrec, roi_level


def _make_sc_call(N, C):
    CCH = C // LANES
    BPW = (N + NW - 1) // NW
    mesh = plsc.VectorSubcoreMesh(
        core_axis_name="c", subcore_axis_name="s", num_cores=NC, num_subcores=NS
    )

    @functools.partial(
        pl.kernel,
        out_type=jax.ShapeDtypeStruct((N * NSAMP, C), jnp.float32),
        mesh=mesh,
        compiler_params=pltpu.CompilerParams(use_tc_tiling_on_sc=False),
        scratch_types=[
            pltpu.VMEM((BPW + LANES,), jnp.int32),
            pltpu.VMEM((BPW, 4, NSAMP), jnp.int32),
            pltpu.VMEM((BPW, 2 * KPAD), jnp.float32),
            pltpu.VMEM((2, 4, NSAMP, C), jnp.float32),
            pltpu.VMEM((NSAMP, C), jnp.float32),
            pltpu.SemaphoreType.DMA((2,)),
            pltpu.SemaphoreType.DMA,
        ],
    )
    def roialign_sc(t2, t3, t4, t5, idx_hbm, w_hbm, lvl_hbm, out_hbm,
                    lvl_v, idx_all, w_all, rows2, out_v, sem_g, sem_out):
        wid = lax.axis_index("s") * NC + lax.axis_index("c")
        start = wid * BPW
        nb = jnp.clip(N - start, 0, BPW)
        pltpu.sync_copy(lvl_hbm.at[pl.ds(start, BPW + LANES)], lvl_v)
        pltpu.sync_copy(idx_hbm.at[pl.ds(start, BPW)], idx_all)
        pltpu.sync_copy(w_hbm.at[pl.ds(start, BPW)], w_all)

        def fire(bb, mp):
            lv = lvl_v[pl.ds(bb, LANES)][0]
            for level, tref in ((2, t2), (3, t3), (4, t4), (5, t5)):
                def issue(tref=tref):
                    for k in range(4):
                        pltpu.async_copy(
                            tref.at[idx_all.at[bb, k]], rows2.at[mp, k],
                            sem_g.at[mp])
                pl.when(lv == level)(issue)

        def drain(mp):
            for k in range(4):
                pltpu.make_async_copy(
                    t2.at[pl.ds(0, NSAMP)], rows2.at[mp, k], sem_g.at[mp]
                ).wait()

        def wait_out():
            pltpu.make_async_copy(
                out_v, out_hbm.at[pl.ds(0, NSAMP)], sem_out
            ).wait()

        def compute_and_ship(bb, mp):
            def samp(j, c2):
                ly_s = w_all[bb, pl.ds(j, LANES)][0]
                lx_s = w_all[bb, pl.ds(KPAD + j, LANES)][0]
                for c in range(CCH):
                    s = pl.ds(c * LANES, LANES)
                    v00 = rows2[mp, 0, j, s]
                    v01 = rows2[mp, 1, j, s]
                    v10 = rows2[mp, 2, j, s]
                    v11 = rows2[mp, 3, j, s]
                    top = v00 + (v01 - v00) * lx_s
                    bot = v10 + (v11 - v10) * lx_s
                    out_v[j, s] = top + (bot - top) * ly_s
                return c2

            lax.fori_loop(0, NSAMP, samp, 0)
            pltpu.async_copy(
                out_v, out_hbm.at[pl.ds((start + bb) * NSAMP, NSAMP)], sem_out)

        pl.when(nb > 0)(lambda: fire(0, 0))

        def pair_body(p, carry):
            b0 = 2 * p
            b1 = b0 + 1

            pl.when(b1 < nb)(lambda: fire(b1, 1))
            drain(0)
            pl.when(b0 > 0)(wait_out)
            compute_and_ship(b0, 0)

            def do_b1():
                pl.when(b0 + 2 < nb)(lambda: fire(b0 + 2, 0))
                drain(1)
                wait_out()
                compute_and_ship(b1, 1)

            pl.when(b1 < nb)(do_b1)
            return carry

        lax.fori_loop(0, (nb + 1) // 2, pair_body, 0)
        pl.when(nb > 0)(wait_out)

    return roialign_sc


def kernel(boxes, image_shape, P2, P3, P4, P5):
    B, N = boxes.shape[0], boxes.shape[1]
    C = P2.shape[-1]
    sizes = (P2.shape[1], P3.shape[1], P4.shape[1], P5.shape[1])
    idx4, wrec, lvl = _prep(boxes, image_shape, sizes)
    BPW = (B * N + NW - 1) // NW
    nslots = NW * BPW
    idx_pad = jnp.pad(idx4, ((0, nslots - B * N), (0, 0), (0, 0)))
    w_pad = jnp.pad(wrec, ((0, nslots - B * N), (0, 0)))
    lvl_pad = jnp.pad(lvl, (0, nslots + LANES - B * N), constant_values=2)
    tables = [p.reshape(-1, C) for p in (P2, P3, P4, P5)]
    out = _make_sc_call(B * N, C)(*tables, idx_pad, w_pad, lvl_pad)
    return out.reshape(B, N, POOL, POOL, C)
